# baseline (speedup 1.0000x reference)
import functools

import jax
import jax.numpy as jnp
from jax import lax
from jax.experimental import pallas as pl
from jax.experimental.pallas import tpu as pltpu
from jax.experimental.pallas import tpu_sc as plsc

N = 10000
E = 320000
D = 128
H = 128
C = 47

NC = 2
NS = 16
NW = NC * NS
CHUNK = 128
NBUF = 3
PC = 79
EPAD = NW * CHUNK * PC
RPS = 632
NPAD = 10104
LRPS = NPAD - (NS - 1) * RPS


def _agg_sc(p, src_b, dst_b, width):
    mesh = plsc.VectorSubcoreMesh(core_axis_name="c", subcore_axis_name="s")
    scratch = [
        pltpu.VMEM((8, CHUNK), jnp.int32),
        pltpu.VMEM((NBUF * CHUNK, width), jnp.float32),
        pltpu.VMEM_SHARED((NPAD, width), jnp.float32),
        pltpu.SemaphoreType.DMA((NBUF,)),
        pltpu.SemaphoreType.DMA((NBUF,)),
        pltpu.SemaphoreType.DMA((NBUF,)),
    ]

    @functools.partial(
        pl.kernel, mesh=mesh,
        out_type=jax.ShapeDtypeStruct((NC, NPAD, width), jnp.float32),
        scratch_types=scratch)
    def k(p_hbm, src_hbm, dst_hbm, z_hbm, out_hbm, idx_v, ring, acc,
          isem, gsem, ssem):
        c = lax.axis_index("c")
        s = lax.axis_index("s")
        wid = c * NS + s
        base = s * RPS
        @pl.when(s < NS - 1)
        def _():
            pltpu.sync_copy(z_hbm, acc.at[pl.ds(base, RPS)])
        @pl.when(s == NS - 1)
        def _():
            pltpu.sync_copy(z_hbm.at[pl.ds(0, LRPS)], acc.at[pl.ds(base, LRPS)])
        plsc.subcore_barrier()

        def i_descs(j, b):
            return (pltpu.make_async_copy(src_hbm.at[wid * PC + j],
                                          idx_v.at[b], isem.at[b]),
                    pltpu.make_async_copy(dst_hbm.at[wid * PC + j],
                                          idx_v.at[4 + b], isem.at[b]))
        def g_desc(j, b):
            return pltpu.make_async_copy(
                p_hbm.at[idx_v.at[b]],
                ring.at[pl.ds(b * CHUNK, CHUNK)], gsem.at[b])
        def s_desc(j, b):
            return pltpu.make_async_copy(
                ring.at[pl.ds(b * CHUNK, CHUNK)],
                acc.at[idx_v.at[4 + b]], ssem.at[b])

        @pl.loop(0, PC + 2)
        def _(t):
            @pl.when(t >= 2)
            def _():
                j = t - 2
                b = lax.rem(j, NBUF)
                g_desc(j, b).wait()
                pltpu.async_copy(ring.at[pl.ds(b * CHUNK, CHUNK)],
                                 acc.at[idx_v.at[4 + b]], ssem.at[b],
                                 add=True)
            @pl.when(t < PC)
            def _():
                b = lax.rem(t, NBUF)
                @pl.when(t >= NBUF)
                def _():
                    s_desc(t - NBUF, b).wait()
                d0, d1 = i_descs(t, b)
                d0.start()
                d1.start()
            @pl.when(jnp.logical_and(t >= 1, t <= PC))
            def _():
                j = t - 1
                b = lax.rem(j, NBUF)
                d0, d1 = i_descs(j, b)
                d0.wait()
                d1.wait()
                g_desc(j, b).start()
        @pl.loop(PC - NBUF, PC)
        def _(j):
            s_desc(j, lax.rem(j, NBUF)).wait()

        plsc.subcore_barrier()
        @pl.when(s < NS - 1)
        def _():
            pltpu.sync_copy(acc.at[pl.ds(base, RPS)],
                            out_hbm.at[c, pl.ds(base, RPS)])
        @pl.when(s == NS - 1)
        def _():
            pltpu.sync_copy(acc.at[pl.ds(base, LRPS)],
                            out_hbm.at[c, pl.ds(base, LRPS)])

    z = jnp.zeros((RPS, width), jnp.float32)
    return k(p, src_b.reshape(NW * PC, CHUNK), dst_b.reshape(NW * PC, CHUNK), z)


def _deg_sc(dst_b):
    mesh = plsc.VectorSubcoreMesh(core_axis_name="c", subcore_axis_name="s")
    scratch = [
        pltpu.VMEM((PC, CHUNK), jnp.int32),
        pltpu.VMEM((CHUNK, H), jnp.float32),
        pltpu.VMEM_SHARED((NPAD, H), jnp.float32),
        pltpu.SemaphoreType.DMA,
    ]

    @functools.partial(
        pl.kernel, mesh=mesh,
        out_type=jax.ShapeDtypeStruct((NC, NPAD, H), jnp.float32),
        scratch_types=scratch)
    def k(dst_hbm, z_hbm, ones_hbm, cnt_hbm, dst_v, ones_v, acc, sem):
        c = lax.axis_index("c")
        s = lax.axis_index("s")
        wid = c * NS + s
        base = s * RPS
        pltpu.sync_copy(dst_hbm.at[wid], dst_v)
        pltpu.sync_copy(ones_hbm, ones_v)
        @pl.when(s < NS - 1)
        def _():
            pltpu.sync_copy(z_hbm, acc.at[pl.ds(base, RPS)])
        @pl.when(s == NS - 1)
        def _():
            pltpu.sync_copy(z_hbm.at[pl.ds(0, LRPS)], acc.at[pl.ds(base, LRPS)])
        plsc.subcore_barrier()
        DEPTH = 4
        for j in range(DEPTH):
            pltpu.async_copy(ones_v, acc.at[dst_v.at[j]], sem, add=True)
        @pl.loop(DEPTH, PC)
        def _(j):
            pltpu.make_async_copy(ones_v, acc.at[dst_v.at[j - DEPTH]],
                                  sem).wait()
            pltpu.async_copy(ones_v, acc.at[dst_v.at[j]], sem, add=True)
        @pl.loop(PC - DEPTH, PC)
        def _(j):
            pltpu.make_async_copy(ones_v, acc.at[dst_v.at[j]], sem).wait()
        plsc.subcore_barrier()
        @pl.when(s < NS - 1)
        def _():
            pltpu.sync_copy(acc.at[pl.ds(base, RPS)],
                            cnt_hbm.at[c, pl.ds(base, RPS)])
        @pl.when(s == NS - 1)
        def _():
            pltpu.sync_copy(acc.at[pl.ds(base, LRPS)],
                            cnt_hbm.at[c, pl.ds(base, LRPS)])

    z = jnp.zeros((RPS, H), jnp.float32)
    ones = jnp.ones((CHUNK, H), jnp.float32)
    return k(dst_b, z, ones)


_PREC = lax.Precision.HIGHEST


def _prep0(x, Ws, Wn, b):
    def body(x_ref, ws_ref, wn_ref, b_ref, s_ref, p_ref):
        xv = x_ref[...]
        s_ref[...] = jnp.dot(xv, ws_ref[...], precision=_PREC) + b_ref[...]
        p_ref[...] = jnp.dot(xv, wn_ref[...], precision=_PREC)
    return pl.pallas_call(
        body,
        out_shape=(jax.ShapeDtypeStruct((N, H), jnp.float32),
                   jax.ShapeDtypeStruct((N, H), jnp.float32)),
    )(x, Ws, Wn, b.reshape(1, H))


def _combine(s_prev, part, cnt, Ws, Wn, b):
    def body(s_ref, part_ref, cnt_ref, ws_ref, wn_ref, b_ref, s_ref_o, p_ref_o):
        agg = part_ref[0, :N, :] + part_ref[1, :N, :]
        deg = cnt_ref[0, :N, 0:1] + cnt_ref[1, :N, 0:1]
        inv = 1.0 / jnp.maximum(deg, 1.0)
        h = jax.nn.relu(s_ref[...] + agg * inv)
        s_ref_o[...] = jnp.dot(h, ws_ref[...], precision=_PREC) + b_ref[...]
        p_ref_o[...] = jnp.dot(h, wn_ref[...], precision=_PREC)
    ow = Ws.shape[1]
    return pl.pallas_call(
        body,
        out_shape=(jax.ShapeDtypeStruct((N, ow), jnp.float32),
                   jax.ShapeDtypeStruct((N, ow), jnp.float32)),
    )(s_prev, part, cnt, Ws, Wn, b.reshape(1, ow))


def _final(s_prev, part, cnt):
    def body(s_ref, part_ref, cnt_ref, out_ref):
        agg = part_ref[0, :N, :] + part_ref[1, :N, :]
        deg = cnt_ref[0, :N, 0:1] + cnt_ref[1, :N, 0:1]
        inv = 1.0 / jnp.maximum(deg, 1.0)
        out_ref[...] = (s_ref[...] + agg * inv)[:, :C]
    return pl.pallas_call(
        body,
        out_shape=jax.ShapeDtypeStruct((N, C), jnp.float32),
    )(s_prev, part, cnt)


def kernel(x, edge_index, W_self0, W_neigh0, b0, W_self1, W_neigh1, b1,
           W_self2, W_neigh2, b2):
    src = edge_index[0]
    dst = edge_index[1]
    src_s, dst_s = jax.lax.sort_key_val(src, dst)
    pad = EPAD - E
    src_b = jnp.concatenate([src_s, jnp.zeros((pad,), jnp.int32)]).reshape(NW, PC, CHUNK)
    dst_b = jnp.concatenate([dst_s, jnp.full((pad,), N, jnp.int32)]).reshape(NW, PC, CHUNK)

    cnt = _deg_sc(dst_b)[:, :, :16]
    s0, p0 = _prep0(x, W_self0, W_neigh0, b0)
    part0 = _agg_sc(p0, src_b, dst_b, H)
    s1, p1 = _combine(s0, part0, cnt, W_self1, W_neigh1, b1)
    part1 = _agg_sc(p1, src_b, dst_b, H)
    Ws2 = jnp.pad(W_self2, ((0, 0), (0, H - C)))
    Wn2 = jnp.pad(W_neigh2, ((0, 0), (0, H - C)))
    b2p = jnp.pad(b2, (0, H - C))
    s2, p2 = _combine(s1, part1, cnt, Ws2, Wn2, b2p)
    part2 = _agg_sc(p2, src_b, dst_b, H)
    return _final(s2, part2, cnt)


# speedup vs baseline: 1.7731x; 1.7731x over previous
import functools

import jax
import jax.numpy as jnp
from jax import lax
from jax.experimental import pallas as pl
from jax.experimental.pallas import tpu as pltpu
from jax.experimental.pallas import tpu_sc as plsc

N = 10000
E = 320000
D = 128
H = 128
C = 47

NC = 2
NS = 16
NW = NC * NS
CHUNK = 128
NBUF = 3
PC = 79
EPAD = NW * CHUNK * PC
RPS = 632
NPAD = 10104
LRPS = NPAD - (NS - 1) * RPS


def _agg_sc(p, src_b, dst_b, width):
    mesh = plsc.VectorSubcoreMesh(core_axis_name="c", subcore_axis_name="s")
    scratch = [
        pltpu.VMEM((8, CHUNK), jnp.int32),
        pltpu.VMEM((NBUF * CHUNK, width), jnp.float32),
        pltpu.VMEM_SHARED((NPAD, width), jnp.float32),
        pltpu.SemaphoreType.DMA((NBUF,)),
        pltpu.SemaphoreType.DMA((NBUF,)),
        pltpu.SemaphoreType.DMA((NBUF,)),
    ]

    @functools.partial(
        pl.kernel, mesh=mesh,
        out_type=jax.ShapeDtypeStruct((NC, NPAD, width), jnp.float32),
        scratch_types=scratch)
    def k(p_hbm, src_hbm, dst_hbm, z_hbm, out_hbm, idx_v, ring, acc,
          isem, gsem, ssem):
        c = lax.axis_index("c")
        s = lax.axis_index("s")
        wid = c * NS + s
        base = s * RPS
        @pl.when(s < NS - 1)
        def _():
            pltpu.sync_copy(z_hbm, acc.at[pl.ds(base, RPS)])
        @pl.when(s == NS - 1)
        def _():
            pltpu.sync_copy(z_hbm.at[pl.ds(0, LRPS)], acc.at[pl.ds(base, LRPS)])
        plsc.subcore_barrier()

        def i_descs(j, b):
            return (pltpu.make_async_copy(src_hbm.at[wid * PC + j],
                                          idx_v.at[b], isem.at[b]),
                    pltpu.make_async_copy(dst_hbm.at[wid * PC + j],
                                          idx_v.at[4 + b], isem.at[b]))
        def g_desc(j, b):
            return pltpu.make_async_copy(
                p_hbm.at[idx_v.at[b]],
                ring.at[pl.ds(b * CHUNK, CHUNK)], gsem.at[b])
        def s_desc(j, b):
            return pltpu.make_async_copy(
                ring.at[pl.ds(b * CHUNK, CHUNK)],
                acc.at[idx_v.at[4 + b]], ssem.at[b])

        @pl.loop(0, PC + 2)
        def _(t):
            @pl.when(t >= 2)
            def _():
                j = t - 2
                b = lax.rem(j, NBUF)
                g_desc(j, b).wait()
                pltpu.async_copy(ring.at[pl.ds(b * CHUNK, CHUNK)],
                                 acc.at[idx_v.at[4 + b]], ssem.at[b],
                                 add=True)
            @pl.when(t < PC)
            def _():
                b = lax.rem(t, NBUF)
                @pl.when(t >= NBUF)
                def _():
                    s_desc(t - NBUF, b).wait()
                d0, d1 = i_descs(t, b)
                d0.start()
                d1.start()
            @pl.when(jnp.logical_and(t >= 1, t <= PC))
            def _():
                j = t - 1
                b = lax.rem(j, NBUF)
                d0, d1 = i_descs(j, b)
                d0.wait()
                d1.wait()
                g_desc(j, b).start()
        @pl.loop(PC - NBUF, PC)
        def _(j):
            s_desc(j, lax.rem(j, NBUF)).wait()

        plsc.subcore_barrier()
        @pl.when(s < NS - 1)
        def _():
            pltpu.sync_copy(acc.at[pl.ds(base, RPS)],
                            out_hbm.at[c, pl.ds(base, RPS)])
        @pl.when(s == NS - 1)
        def _():
            pltpu.sync_copy(acc.at[pl.ds(base, LRPS)],
                            out_hbm.at[c, pl.ds(base, LRPS)])

    z = jnp.zeros((RPS, width), jnp.float32)
    return k(p, src_b.reshape(NW * PC, CHUNK), dst_b.reshape(NW * PC, CHUNK), z)


def _deg_sc(dst_b):
    mesh = plsc.VectorSubcoreMesh(core_axis_name="c", subcore_axis_name="s")
    scratch = [
        pltpu.VMEM((PC, CHUNK), jnp.int32),
        pltpu.VMEM((CHUNK, H), jnp.float32),
        pltpu.VMEM_SHARED((NPAD, H), jnp.float32),
        pltpu.SemaphoreType.DMA,
    ]

    @functools.partial(
        pl.kernel, mesh=mesh,
        out_type=jax.ShapeDtypeStruct((NC, NPAD, H), jnp.float32),
        scratch_types=scratch)
    def k(dst_hbm, z_hbm, ones_hbm, cnt_hbm, dst_v, ones_v, acc, sem):
        c = lax.axis_index("c")
        s = lax.axis_index("s")
        wid = c * NS + s
        base = s * RPS
        pltpu.sync_copy(dst_hbm.at[wid], dst_v)
        pltpu.sync_copy(ones_hbm, ones_v)
        @pl.when(s < NS - 1)
        def _():
            pltpu.sync_copy(z_hbm, acc.at[pl.ds(base, RPS)])
        @pl.when(s == NS - 1)
        def _():
            pltpu.sync_copy(z_hbm.at[pl.ds(0, LRPS)], acc.at[pl.ds(base, LRPS)])
        plsc.subcore_barrier()
        DEPTH = 4
        for j in range(DEPTH):
            pltpu.async_copy(ones_v, acc.at[dst_v.at[j]], sem, add=True)
        @pl.loop(DEPTH, PC)
        def _(j):
            pltpu.make_async_copy(ones_v, acc.at[dst_v.at[j - DEPTH]],
                                  sem).wait()
            pltpu.async_copy(ones_v, acc.at[dst_v.at[j]], sem, add=True)
        @pl.loop(PC - DEPTH, PC)
        def _(j):
            pltpu.make_async_copy(ones_v, acc.at[dst_v.at[j]], sem).wait()
        plsc.subcore_barrier()
        @pl.when(s < NS - 1)
        def _():
            pltpu.sync_copy(acc.at[pl.ds(base, RPS)],
                            cnt_hbm.at[c, pl.ds(base, RPS)])
        @pl.when(s == NS - 1)
        def _():
            pltpu.sync_copy(acc.at[pl.ds(base, LRPS)],
                            cnt_hbm.at[c, pl.ds(base, LRPS)])

    z = jnp.zeros((RPS, H), jnp.float32)
    ones = jnp.ones((CHUNK, H), jnp.float32)
    return k(dst_b, z, ones)


_PREC = lax.Precision.DEFAULT


def _prep0(x, Ws, Wn, b):
    def body(x_ref, ws_ref, wn_ref, b_ref, s_ref, p_ref):
        xv = x_ref[...]
        s_ref[...] = jnp.dot(xv, ws_ref[...], precision=_PREC) + b_ref[...]
        p_ref[...] = jnp.dot(xv, wn_ref[...], precision=_PREC)
    return pl.pallas_call(
        body,
        out_shape=(jax.ShapeDtypeStruct((N, H), jnp.float32),
                   jax.ShapeDtypeStruct((N, H), jnp.float32)),
    )(x, Ws, Wn, b.reshape(1, H))


def _combine(s_prev, part, cnt, Ws, Wn, b):
    def body(s_ref, part_ref, cnt_ref, ws_ref, wn_ref, b_ref, s_ref_o, p_ref_o):
        agg = part_ref[0, :N, :] + part_ref[1, :N, :]
        deg = cnt_ref[0, :N, 0:1] + cnt_ref[1, :N, 0:1]
        inv = 1.0 / jnp.maximum(deg, 1.0)
        h = jax.nn.relu(s_ref[...] + agg * inv)
        s_ref_o[...] = jnp.dot(h, ws_ref[...], precision=_PREC) + b_ref[...]
        p_ref_o[...] = jnp.dot(h, wn_ref[...], precision=_PREC)
    ow = Ws.shape[1]
    return pl.pallas_call(
        body,
        out_shape=(jax.ShapeDtypeStruct((N, ow), jnp.float32),
                   jax.ShapeDtypeStruct((N, ow), jnp.float32)),
    )(s_prev, part, cnt, Ws, Wn, b.reshape(1, ow))


def _final(s_prev, part, cnt):
    def body(s_ref, part_ref, cnt_ref, out_ref):
        agg = part_ref[0, :N, :] + part_ref[1, :N, :]
        deg = cnt_ref[0, :N, 0:1] + cnt_ref[1, :N, 0:1]
        inv = 1.0 / jnp.maximum(deg, 1.0)
        out_ref[...] = (s_ref[...] + agg * inv)[:, :C]
    return pl.pallas_call(
        body,
        out_shape=jax.ShapeDtypeStruct((N, C), jnp.float32),
    )(s_prev, part, cnt)


def kernel(x, edge_index, W_self0, W_neigh0, b0, W_self1, W_neigh1, b1,
           W_self2, W_neigh2, b2):
    src = edge_index[0]
    dst = edge_index[1]
    pad = EPAD - E
    src_b = jnp.concatenate([src, jnp.zeros((pad,), jnp.int32)]).reshape(NW, PC, CHUNK)
    dst_b = jnp.concatenate([dst, jnp.full((pad,), N, jnp.int32)]).reshape(NW, PC, CHUNK)

    cnt = _deg_sc(dst_b)
    s0, p0 = _prep0(x, W_self0, W_neigh0, b0)
    part0 = _agg_sc(p0, src_b, dst_b, H)
    s1, p1 = _combine(s0, part0, cnt, W_self1, W_neigh1, b1)
    part1 = _agg_sc(p1, src_b, dst_b, H)
    Ws2 = jnp.pad(W_self2, ((0, 0), (0, H - C)))
    Wn2 = jnp.pad(W_neigh2, ((0, 0), (0, H - C)))
    b2p = jnp.pad(b2, (0, H - C))
    s2, p2 = _combine(s1, part1, cnt, Ws2, Wn2, b2p)
    part2 = _agg_sc(p2, src_b, dst_b, H)
    return _final(s2, part2, cnt)
